# baseline (speedup 1.0000x reference)
import functools

import jax
import jax.numpy as jnp
from jax import lax
from jax.experimental import pallas as pl
from jax.experimental.pallas import tpu as pltpu
from jax.experimental.pallas import tpu_sc as plsc

VOCAB = 100000
EMBED_DIM = 64
BATCH = 1024
SEQ = 200

_NC = 2
_NS = 16
_NW = _NC * _NS
_N_CHUNKS = 2
_CHUNK = BATCH // _N_CHUNKS
_ROWS_PER_W = _CHUNK // _NW
_C0 = 128
_C1 = SEQ - _C0


def _sc_pool_body(chunk, x_hbm, table_hbm, out_hbm, idx_v, rows_a, rows_b,
                  pooled_v, sem_a, sem_b):
    wid = lax.axis_index("s") * _NC + lax.axis_index("c")
    base = wid * _ROWS_PER_W
    xbase = jnp.int32(chunk * _CHUNK) + base

    pltpu.sync_copy(x_hbm.at[pl.ds(xbase, _ROWS_PER_W)], idx_v)

    inv = jnp.float32(1.0 / SEQ)

    def fire(i, rows, sem):
        pltpu.async_copy(
            table_hbm.at[idx_v.at[i, pl.ds(0, _C0)]],
            rows.at[pl.ds(0, _C0)], sem)
        pltpu.async_copy(
            table_hbm.at[idx_v.at[i, pl.ds(_C0, _C1)]],
            rows.at[pl.ds(_C0, _C1)], sem)

    def drain(rows, sem):
        pltpu.make_async_copy(
            table_hbm.at[pl.ds(0, _C0)], rows.at[pl.ds(0, _C0)], sem).wait()
        pltpu.make_async_copy(
            table_hbm.at[pl.ds(0, _C1)], rows.at[pl.ds(_C0, _C1)], sem).wait()

    def reduce_row(i, rows):
        def acc_body(j, accs):
            a0, a1, a2, a3 = accs
            return (a0 + rows[j, pl.ds(0, 16)],
                    a1 + rows[j, pl.ds(16, 16)],
                    a2 + rows[j, pl.ds(32, 16)],
                    a3 + rows[j, pl.ds(48, 16)])

        z = jnp.zeros((16,), jnp.float32)
        a0, a1, a2, a3 = lax.fori_loop(0, SEQ, acc_body, (z, z, z, z))
        pooled_v[i, pl.ds(0, 16)] = a0 * inv
        pooled_v[i, pl.ds(16, 16)] = a1 * inv
        pooled_v[i, pl.ds(32, 16)] = a2 * inv
        pooled_v[i, pl.ds(48, 16)] = a3 * inv

    fire(jnp.int32(0), rows_a, sem_a)

    def pair_body(p, carry):
        del carry
        ia = jnp.int32(2) * p
        fire(ia + 1, rows_b, sem_b)
        drain(rows_a, sem_a)
        reduce_row(ia, rows_a)

        @pl.when(ia + 2 < _ROWS_PER_W)
        def _():
            fire(ia + 2, rows_a, sem_a)

        drain(rows_b, sem_b)
        reduce_row(ia + 1, rows_b)
        return 0

    lax.fori_loop(jnp.int32(0), jnp.int32(_ROWS_PER_W // 2), pair_body, 0)

    pltpu.sync_copy(pooled_v, out_hbm.at[pl.ds(base, _ROWS_PER_W)])


@functools.cache
def _build_sc_pool(chunk):
    return pl.kernel(
        functools.partial(_sc_pool_body, chunk),
        mesh=plsc.VectorSubcoreMesh(
            core_axis_name="c", subcore_axis_name="s",
            num_cores=_NC, num_subcores=_NS),
        out_type=jax.ShapeDtypeStruct((_CHUNK, EMBED_DIM), jnp.float32),
        scratch_types=[
            pltpu.VMEM((_ROWS_PER_W, SEQ), jnp.int32),
            pltpu.VMEM((SEQ, EMBED_DIM), jnp.float32),
            pltpu.VMEM((SEQ, EMBED_DIM), jnp.float32),
            pltpu.VMEM((_ROWS_PER_W, EMBED_DIM), jnp.float32),
            pltpu.SemaphoreType.DMA,
            pltpu.SemaphoreType.DMA,
        ],
        compiler_params=pltpu.CompilerParams(use_tc_tiling_on_sc=False),
    )


_N_BLK = 6144


def _i32(v):
    return jnp.asarray(v, jnp.int32)


def _mm_body_first(p_ref, w_ref, b_ref, o_ref):
    acc = lax.dot_general(
        p_ref[...], w_ref[...],
        (((1,), (1,)), ((), ())),
        preferred_element_type=jnp.float32)
    o_ref[...] = acc + b_ref[...]


def _mm_body_next(p_ref, w_ref, b_ref, prev_ref, o_ref):
    del prev_ref
    _mm_body_first(p_ref, w_ref, b_ref, o_ref)


def _proj_chunk(chunk, pooled, W, b2d, prev=None):
    grid = (pl.cdiv(VOCAB, _N_BLK),)
    c = chunk
    in_specs = [
        pl.BlockSpec((_CHUNK, EMBED_DIM), lambda j: (_i32(0), _i32(0))),
        pl.BlockSpec((_N_BLK, EMBED_DIM), lambda j: (_i32(j), _i32(0))),
        pl.BlockSpec((1, _N_BLK), lambda j: (_i32(0), _i32(j))),
    ]
    args = [pooled, W, b2d]
    kwargs = {}
    body = _mm_body_first
    if prev is not None:
        in_specs.append(pl.BlockSpec(memory_space=pl.ANY))
        args.append(prev)
        kwargs["input_output_aliases"] = {3: 0}
        body = _mm_body_next
    return pl.pallas_call(
        body,
        grid=grid,
        in_specs=in_specs,
        out_specs=pl.BlockSpec((_CHUNK, _N_BLK), lambda j: (_i32(c), _i32(j))),
        out_shape=jax.ShapeDtypeStruct((BATCH, VOCAB), jnp.float32),
        **kwargs,
    )(*args)


def kernel(x, table, W, b):
    x32 = x.astype(jnp.int32)
    b2d = b.reshape(1, VOCAB)
    pooled = [_build_sc_pool(c)(x32, table) for c in range(_N_CHUNKS)]
    out = _proj_chunk(0, pooled[0], W, b2d)
    for c in range(1, _N_CHUNKS):
        out = _proj_chunk(c, pooled[c], W, b2d, prev=out)
    return out


# speedup vs baseline: 1.0415x; 1.0415x over previous
import functools

import jax
import jax.numpy as jnp
from jax import lax
from jax.experimental import pallas as pl
from jax.experimental.pallas import tpu as pltpu
from jax.experimental.pallas import tpu_sc as plsc

VOCAB = 100000
EMBED_DIM = 64
BATCH = 1024
SEQ = 200

_NC = 2
_NS = 16
_NW = _NC * _NS
_ROWS_PER_W = BATCH // _NW
_C0 = 128
_C1 = SEQ - _C0
_UNROLL = 8


def _sc_pool_body(x_hbm, table_hbm, out_hbm, idx_v, rows_a, rows_b,
                  pooled_v, sem_a, sem_b):
    wid = lax.axis_index("s") * _NC + lax.axis_index("c")
    base = wid * _ROWS_PER_W

    pltpu.sync_copy(x_hbm.at[pl.ds(base, _ROWS_PER_W)], idx_v)

    inv = jnp.float32(1.0 / SEQ)

    def fire(i, rows, sem):
        pltpu.async_copy(
            table_hbm.at[idx_v.at[i, pl.ds(0, _C0)]],
            rows.at[pl.ds(0, _C0)], sem)
        pltpu.async_copy(
            table_hbm.at[idx_v.at[i, pl.ds(_C0, _C1)]],
            rows.at[pl.ds(_C0, _C1)], sem)

    def drain(rows, sem):
        pltpu.make_async_copy(
            table_hbm.at[pl.ds(0, _C0)], rows.at[pl.ds(0, _C0)], sem).wait()
        pltpu.make_async_copy(
            table_hbm.at[pl.ds(0, _C1)], rows.at[pl.ds(_C0, _C1)], sem).wait()

    def reduce_row(i, rows):
        def acc_body(jj, accs):
            a0, a1, a2, a3 = accs
            j = jj * jnp.int32(_UNROLL)
            for u in range(_UNROLL):
                a0 = a0 + rows[j + u, pl.ds(0, 16)]
                a1 = a1 + rows[j + u, pl.ds(16, 16)]
                a2 = a2 + rows[j + u, pl.ds(32, 16)]
                a3 = a3 + rows[j + u, pl.ds(48, 16)]
            return (a0, a1, a2, a3)

        z = jnp.zeros((16,), jnp.float32)
        a0, a1, a2, a3 = lax.fori_loop(
            jnp.int32(0), jnp.int32(SEQ // _UNROLL), acc_body, (z, z, z, z))
        pooled_v[i, pl.ds(0, 16)] = a0 * inv
        pooled_v[i, pl.ds(16, 16)] = a1 * inv
        pooled_v[i, pl.ds(32, 16)] = a2 * inv
        pooled_v[i, pl.ds(48, 16)] = a3 * inv

    fire(jnp.int32(0), rows_a, sem_a)

    def pair_body(p, carry):
        del carry
        ia = jnp.int32(2) * p
        fire(ia + 1, rows_b, sem_b)
        drain(rows_a, sem_a)
        reduce_row(ia, rows_a)

        @pl.when(ia + 2 < _ROWS_PER_W)
        def _():
            fire(ia + 2, rows_a, sem_a)

        drain(rows_b, sem_b)
        reduce_row(ia + 1, rows_b)
        return 0

    lax.fori_loop(jnp.int32(0), jnp.int32(_ROWS_PER_W // 2), pair_body, 0)

    pltpu.sync_copy(pooled_v, out_hbm.at[pl.ds(base, _ROWS_PER_W)])


@functools.cache
def _build_sc_pool():
    return pl.kernel(
        _sc_pool_body,
        mesh=plsc.VectorSubcoreMesh(
            core_axis_name="c", subcore_axis_name="s",
            num_cores=_NC, num_subcores=_NS),
        out_type=jax.ShapeDtypeStruct((BATCH, EMBED_DIM), jnp.float32),
        scratch_types=[
            pltpu.VMEM((_ROWS_PER_W, SEQ), jnp.int32),
            pltpu.VMEM((SEQ, EMBED_DIM), jnp.float32),
            pltpu.VMEM((SEQ, EMBED_DIM), jnp.float32),
            pltpu.VMEM((_ROWS_PER_W, EMBED_DIM), jnp.float32),
            pltpu.SemaphoreType.DMA,
            pltpu.SemaphoreType.DMA,
        ],
        compiler_params=pltpu.CompilerParams(use_tc_tiling_on_sc=False),
    )


_N_BLK = 6144


def _i32(v):
    return jnp.asarray(v, jnp.int32)


def _mm_body(p_ref, w_ref, b_ref, o_ref):
    acc = lax.dot_general(
        p_ref[...], w_ref[...],
        (((1,), (1,)), ((), ())),
        preferred_element_type=jnp.float32)
    o_ref[...] = acc + b_ref[...]


def _projection(pooled, W, b2d):
    grid = (pl.cdiv(VOCAB, _N_BLK),)
    return pl.pallas_call(
        _mm_body,
        grid=grid,
        in_specs=[
            pl.BlockSpec((BATCH, EMBED_DIM), lambda j: (_i32(0), _i32(0))),
            pl.BlockSpec((_N_BLK, EMBED_DIM), lambda j: (_i32(j), _i32(0))),
            pl.BlockSpec((1, _N_BLK), lambda j: (_i32(0), _i32(j))),
        ],
        out_specs=pl.BlockSpec((BATCH, _N_BLK), lambda j: (_i32(0), _i32(j))),
        out_shape=jax.ShapeDtypeStruct((BATCH, VOCAB), jnp.float32),
    )(pooled, W, b2d)


def kernel(x, table, W, b):
    x32 = x.astype(jnp.int32)
    pooled = _build_sc_pool()(x32, table)
    return _projection(pooled, W, b.reshape(1, VOCAB))


# speedup vs baseline: 1.0420x; 1.0005x over previous
import functools

import jax
import jax.numpy as jnp
from jax import lax
from jax.experimental import pallas as pl
from jax.experimental.pallas import tpu as pltpu
from jax.experimental.pallas import tpu_sc as plsc

VOCAB = 100000
EMBED_DIM = 64
BATCH = 1024
SEQ = 200

_NC = 2
_NS = 16
_NW = _NC * _NS
_ROWS_PER_W = BATCH // _NW
_C0 = 128
_C1 = SEQ - _C0
_UNROLL = 8


def _sc_pool_body(x_hbm, table_hbm, out_hbm, idx_v, rows_a, rows_b,
                  pooled_v, sem_a, sem_b):
    wid = lax.axis_index("s") * _NC + lax.axis_index("c")
    base = wid * _ROWS_PER_W

    pltpu.sync_copy(x_hbm.at[pl.ds(base, _ROWS_PER_W)], idx_v)

    inv = jnp.float32(1.0 / SEQ)

    def fire(i, rows, sem):
        for u in range(2):
            pltpu.async_copy(
                table_hbm.at[idx_v.at[i + u, pl.ds(0, _C0)]],
                rows.at[jnp.int32(u)].at[pl.ds(0, _C0)], sem)
            pltpu.async_copy(
                table_hbm.at[idx_v.at[i + u, pl.ds(_C0, _C1)]],
                rows.at[jnp.int32(u)].at[pl.ds(_C0, _C1)], sem)

    def drain(rows, sem):
        for u in range(2):
            pltpu.make_async_copy(
                table_hbm.at[pl.ds(0, _C0)],
                rows.at[jnp.int32(u)].at[pl.ds(0, _C0)], sem).wait()
            pltpu.make_async_copy(
                table_hbm.at[pl.ds(0, _C1)],
                rows.at[jnp.int32(u)].at[pl.ds(_C0, _C1)], sem).wait()

    def reduce_row(i, rows):
        def acc_body(jj, accs):
            a0, a1, a2, a3 = accs
            j = jj * jnp.int32(_UNROLL)
            for u in range(_UNROLL):
                a0 = a0 + rows[j + u, pl.ds(0, 16)]
                a1 = a1 + rows[j + u, pl.ds(16, 16)]
                a2 = a2 + rows[j + u, pl.ds(32, 16)]
                a3 = a3 + rows[j + u, pl.ds(48, 16)]
            return (a0, a1, a2, a3)

        z = jnp.zeros((16,), jnp.float32)
        a0, a1, a2, a3 = lax.fori_loop(
            jnp.int32(0), jnp.int32(SEQ // _UNROLL), acc_body, (z, z, z, z))
        pooled_v[i, pl.ds(0, 16)] = a0 * inv
        pooled_v[i, pl.ds(16, 16)] = a1 * inv
        pooled_v[i, pl.ds(32, 16)] = a2 * inv
        pooled_v[i, pl.ds(48, 16)] = a3 * inv

    fire(jnp.int32(0), rows_a, sem_a)

    def quad_body(p, carry):
        del carry
        ia = jnp.int32(4) * p
        fire(ia + 2, rows_b, sem_b)
        drain(rows_a, sem_a)
        reduce_row(ia, rows_a.at[jnp.int32(0)])
        reduce_row(ia + 1, rows_a.at[jnp.int32(1)])

        @pl.when(ia + 4 < _ROWS_PER_W)
        def _():
            fire(ia + 4, rows_a, sem_a)

        drain(rows_b, sem_b)
        reduce_row(ia + 2, rows_b.at[jnp.int32(0)])
        reduce_row(ia + 3, rows_b.at[jnp.int32(1)])
        return 0

    lax.fori_loop(jnp.int32(0), jnp.int32(_ROWS_PER_W // 4), quad_body, 0)

    pltpu.sync_copy(pooled_v, out_hbm.at[pl.ds(base, _ROWS_PER_W)])


@functools.cache
def _build_sc_pool():
    return pl.kernel(
        _sc_pool_body,
        mesh=plsc.VectorSubcoreMesh(
            core_axis_name="c", subcore_axis_name="s",
            num_cores=_NC, num_subcores=_NS),
        out_type=jax.ShapeDtypeStruct((BATCH, EMBED_DIM), jnp.float32),
        scratch_types=[
            pltpu.VMEM((_ROWS_PER_W, SEQ), jnp.int32),
            pltpu.VMEM((2, SEQ, EMBED_DIM), jnp.float32),
            pltpu.VMEM((2, SEQ, EMBED_DIM), jnp.float32),
            pltpu.VMEM((_ROWS_PER_W, EMBED_DIM), jnp.float32),
            pltpu.SemaphoreType.DMA,
            pltpu.SemaphoreType.DMA,
        ],
        compiler_params=pltpu.CompilerParams(use_tc_tiling_on_sc=False),
    )


_N_BLK = 6144


def _i32(v):
    return jnp.asarray(v, jnp.int32)


def _mm_body(p_ref, w_ref, b_ref, o_ref):
    acc = lax.dot_general(
        p_ref[...], w_ref[...],
        (((1,), (1,)), ((), ())),
        preferred_element_type=jnp.float32)
    o_ref[...] = acc + b_ref[...]


def _projection(pooled, W, b2d):
    grid = (pl.cdiv(VOCAB, _N_BLK),)
    return pl.pallas_call(
        _mm_body,
        grid=grid,
        in_specs=[
            pl.BlockSpec((BATCH, EMBED_DIM), lambda j: (_i32(0), _i32(0))),
            pl.BlockSpec((_N_BLK, EMBED_DIM), lambda j: (_i32(j), _i32(0))),
            pl.BlockSpec((1, _N_BLK), lambda j: (_i32(0), _i32(j))),
        ],
        out_specs=pl.BlockSpec((BATCH, _N_BLK), lambda j: (_i32(0), _i32(j))),
        out_shape=jax.ShapeDtypeStruct((BATCH, VOCAB), jnp.float32),
    )(pooled, W, b2d)


def kernel(x, table, W, b):
    x32 = x.astype(jnp.int32)
    pooled = _build_sc_pool()(x32, table)
    return _projection(pooled, W, b.reshape(1, VOCAB))


# speedup vs baseline: 1.9882x; 1.9080x over previous
import functools

import jax
import jax.numpy as jnp
from jax import lax
from jax.experimental import pallas as pl
from jax.experimental.pallas import tpu as pltpu
from jax.experimental.pallas import tpu_sc as plsc

VOCAB = 100000
EMBED_DIM = 64
BATCH = 1024
SEQ = 200

_NC = 2
_NS = 16
_NW = _NC * _NS
_ROWS_PER_W = BATCH // _NW
_C0 = 128
_C1 = SEQ - _C0
_UNROLL = 8


def _sc_pool_body(x_hbm, table_hbm, out_hbm, idx_v, rows_a, rows_b,
                  pooled_v, sem_a, sem_b):
    wid = lax.axis_index("s") * _NC + lax.axis_index("c")
    base = wid * _ROWS_PER_W

    pltpu.sync_copy(x_hbm.at[pl.ds(base, _ROWS_PER_W)], idx_v)

    inv = jnp.float32(1.0 / SEQ)

    def fire(i, rows, sem):
        for u in range(2):
            pltpu.async_copy(
                table_hbm.at[idx_v.at[i + u, pl.ds(0, _C0)]],
                rows.at[jnp.int32(u)].at[pl.ds(0, _C0)], sem)
            pltpu.async_copy(
                table_hbm.at[idx_v.at[i + u, pl.ds(_C0, _C1)]],
                rows.at[jnp.int32(u)].at[pl.ds(_C0, _C1)], sem)

    def drain(rows, sem):
        for u in range(2):
            pltpu.make_async_copy(
                table_hbm.at[pl.ds(0, _C0)],
                rows.at[jnp.int32(u)].at[pl.ds(0, _C0)], sem).wait()
            pltpu.make_async_copy(
                table_hbm.at[pl.ds(0, _C1)],
                rows.at[jnp.int32(u)].at[pl.ds(_C0, _C1)], sem).wait()

    def reduce_row(i, rows):
        def acc_body(jj, accs):
            a0, a1, a2, a3 = accs
            j = jj * jnp.int32(_UNROLL)
            for u in range(_UNROLL):
                a0 = a0 + rows[j + u, pl.ds(0, 16)]
                a1 = a1 + rows[j + u, pl.ds(16, 16)]
                a2 = a2 + rows[j + u, pl.ds(32, 16)]
                a3 = a3 + rows[j + u, pl.ds(48, 16)]
            return (a0, a1, a2, a3)

        z = jnp.zeros((16,), jnp.float32)
        a0, a1, a2, a3 = lax.fori_loop(
            jnp.int32(0), jnp.int32(SEQ // _UNROLL), acc_body, (z, z, z, z))
        pooled_v[i, pl.ds(0, 16)] = a0 * inv
        pooled_v[i, pl.ds(16, 16)] = a1 * inv
        pooled_v[i, pl.ds(32, 16)] = a2 * inv
        pooled_v[i, pl.ds(48, 16)] = a3 * inv

    fire(jnp.int32(0), rows_a, sem_a)

    def quad_body(p, carry):
        del carry
        ia = jnp.int32(4) * p
        fire(ia + 2, rows_b, sem_b)
        drain(rows_a, sem_a)
        reduce_row(ia, rows_a.at[jnp.int32(0)])
        reduce_row(ia + 1, rows_a.at[jnp.int32(1)])

        @pl.when(ia + 4 < _ROWS_PER_W)
        def _():
            fire(ia + 4, rows_a, sem_a)

        drain(rows_b, sem_b)
        reduce_row(ia + 2, rows_b.at[jnp.int32(0)])
        reduce_row(ia + 3, rows_b.at[jnp.int32(1)])
        return 0

    lax.fori_loop(jnp.int32(0), jnp.int32(_ROWS_PER_W // 4), quad_body, 0)

    pltpu.sync_copy(pooled_v, out_hbm.at[pl.ds(base, _ROWS_PER_W)])


@functools.cache
def _build_sc_pool():
    return pl.kernel(
        _sc_pool_body,
        mesh=plsc.VectorSubcoreMesh(
            core_axis_name="c", subcore_axis_name="s",
            num_cores=_NC, num_subcores=_NS),
        out_type=jax.ShapeDtypeStruct((BATCH, EMBED_DIM), jnp.float32),
        scratch_types=[
            pltpu.VMEM((_ROWS_PER_W, SEQ), jnp.int32),
            pltpu.VMEM((2, SEQ, EMBED_DIM), jnp.float32),
            pltpu.VMEM((2, SEQ, EMBED_DIM), jnp.float32),
            pltpu.VMEM((_ROWS_PER_W, EMBED_DIM), jnp.float32),
            pltpu.SemaphoreType.DMA,
            pltpu.SemaphoreType.DMA,
        ],
        compiler_params=pltpu.CompilerParams(use_tc_tiling_on_sc=False),
    )


_N_BLK = 4096


def _i32(v):
    return jnp.asarray(v, jnp.int32)


def _mm_body(w_ref, p_ref, b_ref, o_ref):
    acc = lax.dot_general(
        w_ref[...], p_ref[...],
        (((1,), (1,)), ((), ())),
        preferred_element_type=jnp.float32)
    o_ref[...] = acc + b_ref[...]


def _projection_t(pooled, W, bcol):
    grid = (pl.cdiv(VOCAB, _N_BLK),)
    return pl.pallas_call(
        _mm_body,
        grid=grid,
        in_specs=[
            pl.BlockSpec((_N_BLK, EMBED_DIM), lambda j: (_i32(j), _i32(0))),
            pl.BlockSpec((BATCH, EMBED_DIM), lambda j: (_i32(0), _i32(0))),
            pl.BlockSpec((_N_BLK, 1), lambda j: (_i32(j), _i32(0))),
        ],
        out_specs=pl.BlockSpec((_N_BLK, BATCH), lambda j: (_i32(j), _i32(0))),
        out_shape=jax.ShapeDtypeStruct((VOCAB, BATCH), jnp.float32),
    )(W, pooled, bcol)


def kernel(x, table, W, b):
    x32 = x.astype(jnp.int32)
    pooled = _build_sc_pool()(x32, table)
    return _projection_t(pooled, W, b.reshape(VOCAB, 1)).T


# speedup vs baseline: 2.3525x; 1.1832x over previous
import functools

import jax
import jax.numpy as jnp
from jax import lax
from jax.experimental import pallas as pl
from jax.experimental.pallas import tpu as pltpu
from jax.experimental.pallas import tpu_sc as plsc

VOCAB = 100000
EMBED_DIM = 64
BATCH = 1024
SEQ = 200

_NC = 2
_NS = 16
_NW = _NC * _NS
_ROWS_PER_W = BATCH // _NW
_C0 = 128
_C1 = SEQ - _C0
_UNROLL = 8


def _sc_pool_body(x_hbm, table_hbm, out_hbm, idx_v, rows_a, rows_b,
                  pooled_v, sem_a, sem_b):
    wid = lax.axis_index("s") * _NC + lax.axis_index("c")
    base = wid * _ROWS_PER_W

    pltpu.sync_copy(x_hbm.at[pl.ds(base, _ROWS_PER_W)], idx_v)

    inv = jnp.float32(1.0 / SEQ)

    def fire(i, rows, sem):
        for u in range(2):
            pltpu.async_copy(
                table_hbm.at[idx_v.at[i + u, pl.ds(0, _C0)]],
                rows.at[jnp.int32(u)].at[pl.ds(0, _C0)], sem)
            pltpu.async_copy(
                table_hbm.at[idx_v.at[i + u, pl.ds(_C0, _C1)]],
                rows.at[jnp.int32(u)].at[pl.ds(_C0, _C1)], sem)

    def drain(rows, sem):
        for u in range(2):
            pltpu.make_async_copy(
                table_hbm.at[pl.ds(0, _C0)],
                rows.at[jnp.int32(u)].at[pl.ds(0, _C0)], sem).wait()
            pltpu.make_async_copy(
                table_hbm.at[pl.ds(0, _C1)],
                rows.at[jnp.int32(u)].at[pl.ds(_C0, _C1)], sem).wait()

    def reduce_row(i, rows):
        def acc_body(jj, accs):
            a0, a1, a2, a3 = accs
            j = jj * jnp.int32(_UNROLL)
            for u in range(_UNROLL):
                a0 = a0 + rows[j + u, pl.ds(0, 16)]
                a1 = a1 + rows[j + u, pl.ds(16, 16)]
                a2 = a2 + rows[j + u, pl.ds(32, 16)]
                a3 = a3 + rows[j + u, pl.ds(48, 16)]
            return (a0, a1, a2, a3)

        z = jnp.zeros((16,), jnp.float32)
        a0, a1, a2, a3 = lax.fori_loop(
            jnp.int32(0), jnp.int32(SEQ // _UNROLL), acc_body, (z, z, z, z))
        pooled_v[i, pl.ds(0, 16)] = a0 * inv
        pooled_v[i, pl.ds(16, 16)] = a1 * inv
        pooled_v[i, pl.ds(32, 16)] = a2 * inv
        pooled_v[i, pl.ds(48, 16)] = a3 * inv

    fire(jnp.int32(0), rows_a, sem_a)

    def quad_body(p, carry):
        del carry
        ia = jnp.int32(4) * p
        fire(ia + 2, rows_b, sem_b)
        drain(rows_a, sem_a)
        reduce_row(ia, rows_a.at[jnp.int32(0)])
        reduce_row(ia + 1, rows_a.at[jnp.int32(1)])

        @pl.when(ia + 4 < _ROWS_PER_W)
        def _():
            fire(ia + 4, rows_a, sem_a)

        drain(rows_b, sem_b)
        reduce_row(ia + 2, rows_b.at[jnp.int32(0)])
        reduce_row(ia + 3, rows_b.at[jnp.int32(1)])
        return 0

    lax.fori_loop(jnp.int32(0), jnp.int32(_ROWS_PER_W // 4), quad_body, 0)

    pltpu.sync_copy(pooled_v, out_hbm.at[pl.ds(base, _ROWS_PER_W)])


@functools.cache
def _build_sc_pool():
    return pl.kernel(
        _sc_pool_body,
        mesh=plsc.VectorSubcoreMesh(
            core_axis_name="c", subcore_axis_name="s",
            num_cores=_NC, num_subcores=_NS),
        out_type=jax.ShapeDtypeStruct((BATCH, EMBED_DIM), jnp.float32),
        scratch_types=[
            pltpu.VMEM((_ROWS_PER_W, SEQ), jnp.int32),
            pltpu.VMEM((2, SEQ, EMBED_DIM), jnp.float32),
            pltpu.VMEM((2, SEQ, EMBED_DIM), jnp.float32),
            pltpu.VMEM((_ROWS_PER_W, EMBED_DIM), jnp.float32),
            pltpu.SemaphoreType.DMA,
            pltpu.SemaphoreType.DMA,
        ],
        compiler_params=pltpu.CompilerParams(use_tc_tiling_on_sc=False),
    )


_N_BLK = 4096


def _i32(v):
    return jnp.asarray(v, jnp.int32)


def _mm_body(wt_ref, p_ref, b_ref, o_ref):
    acc = lax.dot_general(
        wt_ref[...], p_ref[...],
        (((0,), (1,)), ((), ())),
        preferred_element_type=jnp.float32)
    o_ref[...] = acc + b_ref[...]


def _projection_t(pooled, Wt, bcol):
    grid = (pl.cdiv(VOCAB, _N_BLK),)
    return pl.pallas_call(
        _mm_body,
        grid=grid,
        in_specs=[
            pl.BlockSpec((EMBED_DIM, _N_BLK), lambda j: (_i32(0), _i32(j))),
            pl.BlockSpec((BATCH, EMBED_DIM), lambda j: (_i32(0), _i32(0))),
            pl.BlockSpec((_N_BLK, 1), lambda j: (_i32(j), _i32(0))),
        ],
        out_specs=pl.BlockSpec((_N_BLK, BATCH), lambda j: (_i32(j), _i32(0))),
        out_shape=jax.ShapeDtypeStruct((VOCAB, BATCH), jnp.float32),
    )(Wt, pooled, bcol)


def kernel(x, table, W, b):
    x32 = x.astype(jnp.int32)
    pooled = _build_sc_pool()(x32, table)
    return _projection_t(pooled, W.T, b.reshape(VOCAB, 1)).T


# speedup vs baseline: 2.6840x; 1.1409x over previous
import functools

import jax
import jax.numpy as jnp
from jax import lax
from jax.experimental import pallas as pl
from jax.experimental.pallas import tpu as pltpu
from jax.experimental.pallas import tpu_sc as plsc

VOCAB = 100000
EMBED_DIM = 64
BATCH = 1024
SEQ = 200

_NC = 2
_NS = 16
_NW = _NC * _NS
_ROWS_PER_W = BATCH // _NW
_C0 = 128
_C1 = SEQ - _C0
_UNROLL = 8


def _sc_pool_body(x_hbm, table_hbm, out_hbm, idx_v, rows_a, rows_b,
                  pooled_v, sem_a, sem_b):
    wid = lax.axis_index("s") * _NC + lax.axis_index("c")
    base = wid * _ROWS_PER_W

    pltpu.sync_copy(x_hbm.at[pl.ds(base, _ROWS_PER_W)], idx_v)

    inv = jnp.float32(1.0 / SEQ)

    def fire(i, rows, sem):
        for u in range(2):
            pltpu.async_copy(
                table_hbm.at[idx_v.at[i + u, pl.ds(0, _C0)]],
                rows.at[jnp.int32(u)].at[pl.ds(0, _C0)], sem)
            pltpu.async_copy(
                table_hbm.at[idx_v.at[i + u, pl.ds(_C0, _C1)]],
                rows.at[jnp.int32(u)].at[pl.ds(_C0, _C1)], sem)

    def drain(rows, sem):
        for u in range(2):
            pltpu.make_async_copy(
                table_hbm.at[pl.ds(0, _C0)],
                rows.at[jnp.int32(u)].at[pl.ds(0, _C0)], sem).wait()
            pltpu.make_async_copy(
                table_hbm.at[pl.ds(0, _C1)],
                rows.at[jnp.int32(u)].at[pl.ds(_C0, _C1)], sem).wait()

    def reduce_row(i, rows):
        def acc_body(jj, accs):
            a0, a1, a2, a3 = accs
            j = jj * jnp.int32(_UNROLL)
            for u in range(_UNROLL):
                a0 = a0 + rows[j + u, pl.ds(0, 16)]
                a1 = a1 + rows[j + u, pl.ds(16, 16)]
                a2 = a2 + rows[j + u, pl.ds(32, 16)]
                a3 = a3 + rows[j + u, pl.ds(48, 16)]
            return (a0, a1, a2, a3)

        z = jnp.zeros((16,), jnp.float32)
        a0, a1, a2, a3 = lax.fori_loop(
            jnp.int32(0), jnp.int32(SEQ // _UNROLL), acc_body, (z, z, z, z))
        pooled_v[i, pl.ds(0, 16)] = a0 * inv
        pooled_v[i, pl.ds(16, 16)] = a1 * inv
        pooled_v[i, pl.ds(32, 16)] = a2 * inv
        pooled_v[i, pl.ds(48, 16)] = a3 * inv

    fire(jnp.int32(0), rows_a, sem_a)

    def quad_body(p, carry):
        del carry
        ia = jnp.int32(4) * p
        fire(ia + 2, rows_b, sem_b)
        drain(rows_a, sem_a)
        reduce_row(ia, rows_a.at[jnp.int32(0)])
        reduce_row(ia + 1, rows_a.at[jnp.int32(1)])

        @pl.when(ia + 4 < _ROWS_PER_W)
        def _():
            fire(ia + 4, rows_a, sem_a)

        drain(rows_b, sem_b)
        reduce_row(ia + 2, rows_b.at[jnp.int32(0)])
        reduce_row(ia + 3, rows_b.at[jnp.int32(1)])
        return 0

    lax.fori_loop(jnp.int32(0), jnp.int32(_ROWS_PER_W // 4), quad_body, 0)

    pltpu.sync_copy(pooled_v, out_hbm.at[pl.ds(base, _ROWS_PER_W)])


@functools.cache
def _build_sc_pool():
    return pl.kernel(
        _sc_pool_body,
        mesh=plsc.VectorSubcoreMesh(
            core_axis_name="c", subcore_axis_name="s",
            num_cores=_NC, num_subcores=_NS),
        out_type=jax.ShapeDtypeStruct((BATCH, EMBED_DIM), jnp.float32),
        scratch_types=[
            pltpu.VMEM((_ROWS_PER_W, SEQ), jnp.int32),
            pltpu.VMEM((2, SEQ, EMBED_DIM), jnp.float32),
            pltpu.VMEM((2, SEQ, EMBED_DIM), jnp.float32),
            pltpu.VMEM((_ROWS_PER_W, EMBED_DIM), jnp.float32),
            pltpu.SemaphoreType.DMA,
            pltpu.SemaphoreType.DMA,
        ],
        compiler_params=pltpu.CompilerParams(use_tc_tiling_on_sc=False),
    )


_N_BLK = 4096


def _i32(v):
    return jnp.asarray(v, jnp.int32)


def _mm_body(wt_ref, p_ref, b_ref, o_ref):
    acc = lax.dot_general(
        wt_ref[...], p_ref[...],
        (((0,), (1,)), ((), ())),
        preferred_element_type=jnp.float32)
    o_ref[...] = acc + b_ref[...].T


def _projection_t(pooled, Wt, bcol):
    grid = (pl.cdiv(VOCAB, _N_BLK),)
    return pl.pallas_call(
        _mm_body,
        grid=grid,
        in_specs=[
            pl.BlockSpec((EMBED_DIM, _N_BLK), lambda j: (_i32(0), _i32(j))),
            pl.BlockSpec((BATCH, EMBED_DIM), lambda j: (_i32(0), _i32(0))),
            pl.BlockSpec((1, _N_BLK), lambda j: (_i32(0), _i32(j))),
        ],
        out_specs=pl.BlockSpec((_N_BLK, BATCH), lambda j: (_i32(j), _i32(0))),
        out_shape=jax.ShapeDtypeStruct((VOCAB, BATCH), jnp.float32),
    )(Wt, pooled, bcol)


def kernel(x, table, W, b):
    x32 = x.astype(jnp.int32)
    pooled = _build_sc_pool()(x32, table)
    return _projection_t(pooled, W.T, b.reshape(1, VOCAB)).T
